# baseline (speedup 1.0000x reference)
import jax
import jax.numpy as jnp
from jax.experimental import pallas as pl

SZ = 2048
TEMP = 10.0
EPS = 1e-20
NROWS = SZ * SZ * 2 // 128
ROWS_PER_BLOCK = 4096


def _body(g_ref, u_ref, o_ref):
    shape = g_ref.shape
    lane = jax.lax.broadcasted_iota(jnp.int32, shape, 1)
    perm = jnp.where(lane < 64, 2 * lane, 2 * lane - 127)
    gp = jnp.take_along_axis(g_ref[...], perm, axis=1)
    up = jnp.take_along_axis(u_ref[...], perm, axis=1)
    a0 = gp[:, :64]
    a1 = gp[:, 64:]
    u0 = up[:, :64]
    u1 = up[:, 64:]
    L0 = EPS - jnp.log(u0 + EPS)
    L1 = EPS - jnp.log(u1 + EPS)
    lr = jnp.log(L0 / L1)
    s = (a0 - a1 - lr) * (1.0 / TEMP)
    o_ref[...] = 1.0 / (1.0 + jnp.exp(-s))


def kernel(gen_matrix, u):
    gm = gen_matrix.reshape(NROWS, 128)
    uu = u.reshape(NROWS, 128)
    grid = NROWS // ROWS_PER_BLOCK
    out = pl.pallas_call(
        _body,
        grid=(grid,),
        in_specs=[
            pl.BlockSpec((ROWS_PER_BLOCK, 128), lambda i: (i, 0)),
            pl.BlockSpec((ROWS_PER_BLOCK, 128), lambda i: (i, 0)),
        ],
        out_specs=pl.BlockSpec((ROWS_PER_BLOCK, 64), lambda i: (i, 0)),
        out_shape=jax.ShapeDtypeStruct((NROWS, 64), jnp.float32),
    )(gm, uu)
    return out.reshape(SZ, SZ)


# speedup vs baseline: 86.5396x; 86.5396x over previous
import jax
import jax.numpy as jnp
from jax.experimental import pallas as pl

SZ = 2048
TEMP = 10.0
EPS = 1e-20
ROWS_PER_BLOCK = 256


def _native_view(x):
    return (
        x.reshape(SZ, 16, 128, 2)
        .transpose(0, 1, 3, 2)
        .reshape(SZ, 32, 128)
    )


def _body(g_ref, u_ref, o_ref):
    for g in range(16):
        a0 = g_ref[:, 2 * g, :]
        a1 = g_ref[:, 2 * g + 1, :]
        u0 = u_ref[:, 2 * g, :]
        u1 = u_ref[:, 2 * g + 1, :]
        L0 = EPS - jnp.log(u0 + EPS)
        L1 = EPS - jnp.log(u1 + EPS)
        lr = jnp.log(L0 / L1)
        s = (a0 - a1 - lr) * (1.0 / TEMP)
        o_ref[:, 128 * g:128 * (g + 1)] = 1.0 / (1.0 + jnp.exp(-s))


def kernel(gen_matrix, u):
    gm = _native_view(gen_matrix.reshape(SZ, SZ, 2))
    uu = _native_view(u.reshape(SZ, SZ, 2))
    grid = SZ // ROWS_PER_BLOCK
    return pl.pallas_call(
        _body,
        grid=(grid,),
        in_specs=[
            pl.BlockSpec((ROWS_PER_BLOCK, 32, 128), lambda i: (i, 0, 0)),
            pl.BlockSpec((ROWS_PER_BLOCK, 32, 128), lambda i: (i, 0, 0)),
        ],
        out_specs=pl.BlockSpec((ROWS_PER_BLOCK, SZ), lambda i: (i, 0)),
        out_shape=jax.ShapeDtypeStruct((SZ, SZ), jnp.float32),
    )(gm, uu)
